# baseline (speedup 1.0000x reference)
import functools

import jax
import jax.numpy as jnp
import numpy as np
from jax import lax
from jax.experimental import pallas as pl
from jax.experimental.pallas import tpu as pltpu
from jax.experimental.pallas import tpu_sc as plsc

NT = 512
NX = 512
RANK = 64
BATCH = 64
NPTS = 4096
NPTS_TOT = BATCH * NPTS
NW = 32
PER_W = NPTS_TOT // NW
C = 128
NCHUNK = PER_W // C
L = 16

_F32_HI_T = np.float32(NT - 1.001)
_F32_HI_X = np.float32(NX - 1.001)


def _body(y_hbm, table_hbm, tg_hbm, xg_hbm, out_hbm,
          g_v, yc_v, idx_v, w_v, rows_v, out_v, gsem0, gsem1, osem0, osem1):
    wid = lax.axis_index("s") * 2 + lax.axis_index("c")
    base_pt = wid * PER_W

    pltpu.sync_copy(tg_hbm.at[pl.ds(0, L)], g_v.at[0])
    pltpu.sync_copy(tg_hbm.at[pl.ds(NT - L, L)], g_v.at[1])
    pltpu.sync_copy(xg_hbm.at[pl.ds(0, L)], g_v.at[2])
    pltpu.sync_copy(xg_hbm.at[pl.ds(NX - L, L)], g_v.at[3])
    t_min = g_v[0, :][0]
    t_den = g_v[1, :][L - 1] - t_min + np.float32(1e-8)
    x_min = g_v[2, :][0]
    x_den = g_v[3, :][L - 1] - x_min + np.float32(1e-8)

    def stage(g, b, gsem):
        gg = base_pt + g * C
        bi = lax.shift_right_logical(gg, 12)
        ci = lax.bitwise_and(lax.shift_right_logical(gg, 7), (NPTS // C) - 1)
        pltpu.sync_copy(y_hbm.at[bi, ci, 0], yc_v.at[b, 0])
        pltpu.sync_copy(y_hbm.at[bi, ci, 1], yc_v.at[b, 1])
        for v in range(C // L):
            sl = pl.ds(v * L, L)
            tt = yc_v[b, 0, sl]
            xx = yc_v[b, 1, sl]
            ti = (tt - t_min) / t_den * np.float32(NT - 1)
            xi = (xx - x_min) / x_den * np.float32(NX - 1)
            ti = jnp.clip(ti, np.float32(0.0), _F32_HI_T)
            xi = jnp.clip(xi, np.float32(0.0), _F32_HI_X)
            i0 = jnp.clip(ti.astype(jnp.int32), 0, NT - 2)
            j0 = jnp.clip(xi.astype(jnp.int32), 0, NX - 2)
            w_v[b, 0, sl] = ti - i0.astype(jnp.float32)
            w_v[b, 1, sl] = xi - j0.astype(jnp.float32)
            cell = i0 * NX + j0
            idx_v[b, 0, sl] = cell
            idx_v[b, 1, sl] = cell + 1
            idx_v[b, 2, sl] = cell + NX
            idx_v[b, 3, sl] = cell + NX + 1
        for k in range(4):
            pltpu.async_copy(table_hbm.at[idx_v.at[b, k]], rows_v.at[b, k], gsem)

    def drain_gathers(b, gsem):
        for k in range(4):
            pltpu.make_async_copy(
                table_hbm.at[idx_v.at[b, k]], rows_v.at[b, k], gsem).wait()

    lane = lax.iota(jnp.int32, L)
    rvecs = [lane + q * L for q in range(RANK // L)]

    def combine(g, b):
        def grp(v):
            wtv = w_v[b, 0, pl.ds(v * L, L)]
            wxv = w_v[b, 1, pl.ds(v * L, L)]
            for j in range(L):
                p = v * L + j
                wt = wtv[j]
                wx = wxv[j]
                pvec = lane * 0 + p
                for q in range(RANK // L):
                    sl = pl.ds(q * L, L)
                    v00 = rows_v[b, 0, p, sl]
                    v01 = rows_v[b, 1, p, sl]
                    v10 = rows_v[b, 2, p, sl]
                    v11 = rows_v[b, 3, p, sl]
                    v0 = v00 + (v01 - v00) * wx
                    v1 = v10 + (v11 - v10) * wx
                    plsc.store_scatter(out_v.at[b], [rvecs[q], pvec],
                                       v0 + (v1 - v0) * wt)

        plsc.parallel_loop(0, C // L, 1, unroll=1)(grp)

    def out_copies(g, b, osem, issue):
        gg = base_pt + g * C
        bi = lax.shift_right_logical(gg, 12)
        pc = lax.bitwise_and(lax.shift_right_logical(gg, 7), (NPTS // C) - 1)
        for rg in range(RANK // 8):
            src = out_v.at[b, pl.ds(rg * 8, 8), pl.ds(0, C)]
            dst = out_hbm.at[bi, rg, pc]
            if issue:
                pltpu.async_copy(src, dst, osem)
            else:
                pltpu.make_async_copy(src, dst, osem).wait()

    stage(0, 0, gsem0)
    stage(1, 1, gsem1)

    def pair(u, carry):
        for par, (b, gsem, osem) in enumerate(
                ((0, gsem0, osem0), (1, gsem1, osem1))):
            g = 2 * u + par
            drain_gathers(b, gsem)

            @pl.when(u > 0)
            def _():
                out_copies(g - 2, b, osem, issue=False)

            combine(g, b)
            out_copies(g, b, osem, issue=True)

            @pl.when(g + 2 < NCHUNK)
            def _():
                stage(g + 2, b, gsem)
        return carry

    lax.fori_loop(0, NCHUNK // 2, pair, 0)

    out_copies(NCHUNK - 2, 0, osem0, issue=False)
    out_copies(NCHUNK - 1, 1, osem1, issue=False)


@jax.jit
def _run(y_sep, table, t_grid, x_grid):
    mesh = plsc.VectorSubcoreMesh(core_axis_name="c", subcore_axis_name="s")
    f = functools.partial(
        pl.kernel,
        mesh=mesh,
        compiler_params=pltpu.CompilerParams(
            use_tc_tiling_on_sc=False, needs_layout_passes=False),
        out_type=jax.ShapeDtypeStruct(
            (BATCH, RANK // 8, NPTS // C, 8, C), jnp.float32),
        scratch_types=[
            pltpu.VMEM((4, L), jnp.float32),
            pltpu.VMEM((2, 2, C), jnp.float32),
            pltpu.VMEM((2, 4, C), jnp.int32),
            pltpu.VMEM((2, 2, C), jnp.float32),
            pltpu.VMEM((2, 4, C, RANK), jnp.float32),
            pltpu.VMEM((2, RANK, C + 1), jnp.float32),
            pltpu.SemaphoreType.DMA,
            pltpu.SemaphoreType.DMA,
            pltpu.SemaphoreType.DMA,
            pltpu.SemaphoreType.DMA,
        ],
    )(_body)
    return f(y_sep, table, t_grid, x_grid)


def kernel(y, basis_2d, t_grid, x_grid):
    b, n = y.shape[0], y.shape[1]
    y_sep = y.reshape(b, n // C, C, 2).transpose(0, 1, 3, 2)
    out = _run(y_sep, basis_2d.reshape(NT * NX, RANK), t_grid, x_grid)
    return out.transpose(0, 2, 4, 1, 3).reshape(b, n, RANK)


# speedup vs baseline: 1.6106x; 1.6106x over previous
import functools

import jax
import jax.numpy as jnp
import numpy as np
from jax import lax
from jax.experimental import pallas as pl
from jax.experimental.pallas import tpu as pltpu
from jax.experimental.pallas import tpu_sc as plsc

NT = 512
NX = 512
RANK = 64
BATCH = 64
NPTS = 4096
NPTS_TOT = BATCH * NPTS
NW = 32
PER_W = NPTS_TOT // NW
C = 128
NCHUNK = PER_W // C
L = 16

_F32_HI_T = np.float32(NT - 1.001)
_F32_HI_X = np.float32(NX - 1.001)


def _body(y_hbm, table_hbm, tg_hbm, xg_hbm, out_hbm,
          g_v, yc_v, idx_v, w_v, rows_v, out_v, gsem0, gsem1, osem0, osem1):
    wid = lax.axis_index("s") * 2 + lax.axis_index("c")
    base_pt = wid * PER_W

    pltpu.sync_copy(tg_hbm.at[pl.ds(0, L)], g_v.at[0])
    pltpu.sync_copy(tg_hbm.at[pl.ds(NT - L, L)], g_v.at[1])
    pltpu.sync_copy(xg_hbm.at[pl.ds(0, L)], g_v.at[2])
    pltpu.sync_copy(xg_hbm.at[pl.ds(NX - L, L)], g_v.at[3])
    t_min = g_v[0, :][0]
    t_den = g_v[1, :][L - 1] - t_min + np.float32(1e-8)
    x_min = g_v[2, :][0]
    x_den = g_v[3, :][L - 1] - x_min + np.float32(1e-8)

    def stage(g, b, gsem):
        gg = base_pt + g * C
        bi = lax.shift_right_logical(gg, 12)
        ci = lax.bitwise_and(lax.shift_right_logical(gg, 7), (NPTS // C) - 1)
        pltpu.sync_copy(y_hbm.at[bi, ci, 0], yc_v.at[b, 0])
        pltpu.sync_copy(y_hbm.at[bi, ci, 1], yc_v.at[b, 1])
        for v in range(C // L):
            sl = pl.ds(v * L, L)
            tt = yc_v[b, 0, sl]
            xx = yc_v[b, 1, sl]
            ti = (tt - t_min) / t_den * np.float32(NT - 1)
            xi = (xx - x_min) / x_den * np.float32(NX - 1)
            ti = jnp.clip(ti, np.float32(0.0), _F32_HI_T)
            xi = jnp.clip(xi, np.float32(0.0), _F32_HI_X)
            i0 = jnp.clip(ti.astype(jnp.int32), 0, NT - 2)
            j0 = jnp.clip(xi.astype(jnp.int32), 0, NX - 2)
            w_v[b, 0, sl] = ti - i0.astype(jnp.float32)
            w_v[b, 1, sl] = xi - j0.astype(jnp.float32)
            cell = i0 * NX + j0
            idx_v[b, 0, sl] = cell
            idx_v[b, 1, sl] = cell + 1
            idx_v[b, 2, sl] = cell + NX
            idx_v[b, 3, sl] = cell + NX + 1
        for k in range(4):
            pltpu.async_copy(table_hbm.at[idx_v.at[b, k]], rows_v.at[b, k], gsem)

    def drain_gathers(b, gsem):
        for k in range(4):
            pltpu.make_async_copy(
                table_hbm.at[idx_v.at[b, k]], rows_v.at[b, k], gsem).wait()

    def combine(g, b):
        def grp(v):
            wtv = w_v[b, 0, pl.ds(v * L, L)]
            wxv = w_v[b, 1, pl.ds(v * L, L)]
            for j in range(L):
                p = v * L + j
                wt = wtv[j]
                wx = wxv[j]
                for q in range(RANK // L):
                    sl = pl.ds(q * L, L)
                    v00 = rows_v[b, 0, p, sl]
                    v01 = rows_v[b, 1, p, sl]
                    v10 = rows_v[b, 2, p, sl]
                    v11 = rows_v[b, 3, p, sl]
                    v0 = v00 + (v01 - v00) * wx
                    v1 = v10 + (v11 - v10) * wx
                    out_v[b, p, sl] = v0 + (v1 - v0) * wt

        plsc.parallel_loop(0, C // L, 1, unroll=2)(grp)

    def out_slice(g):
        return out_hbm.at[pl.ds(base_pt + g * C, C), pl.ds(0, RANK)]

    stage(0, 0, gsem0)
    stage(1, 1, gsem1)

    def pair(u, carry):
        for par, (b, gsem, osem) in enumerate(
                ((0, gsem0, osem0), (1, gsem1, osem1))):
            g = 2 * u + par
            drain_gathers(b, gsem)

            @pl.when(u > 0)
            def _():
                pltpu.make_async_copy(out_v.at[b], out_slice(g - 2), osem).wait()

            combine(g, b)
            pltpu.async_copy(out_v.at[b], out_slice(g), osem)

            @pl.when(g + 2 < NCHUNK)
            def _():
                stage(g + 2, b, gsem)
        return carry

    lax.fori_loop(0, NCHUNK // 2, pair, 0)

    pltpu.make_async_copy(out_v.at[0], out_slice(NCHUNK - 2), osem0).wait()
    pltpu.make_async_copy(out_v.at[1], out_slice(NCHUNK - 1), osem1).wait()


@jax.jit
def _run(y_sep, table, t_grid, x_grid):
    mesh = plsc.VectorSubcoreMesh(core_axis_name="c", subcore_axis_name="s")
    f = functools.partial(
        pl.kernel,
        mesh=mesh,
        compiler_params=pltpu.CompilerParams(
            use_tc_tiling_on_sc=False, needs_layout_passes=False),
        out_type=jax.ShapeDtypeStruct((NPTS_TOT, 2 * RANK), jnp.float32),
        scratch_types=[
            pltpu.VMEM((4, L), jnp.float32),
            pltpu.VMEM((2, 2, C), jnp.float32),
            pltpu.VMEM((2, 4, C), jnp.int32),
            pltpu.VMEM((2, 2, C), jnp.float32),
            pltpu.VMEM((2, 4, C, RANK), jnp.float32),
            pltpu.VMEM((2, C, RANK), jnp.float32),
            pltpu.SemaphoreType.DMA,
            pltpu.SemaphoreType.DMA,
            pltpu.SemaphoreType.DMA,
            pltpu.SemaphoreType.DMA,
        ],
    )(_body)
    return f(y_sep, table, t_grid, x_grid)


def kernel(y, basis_2d, t_grid, x_grid):
    b, n = y.shape[0], y.shape[1]
    y_sep = y.reshape(b, n // C, C, 2).transpose(0, 1, 3, 2)
    out = _run(y_sep, basis_2d.reshape(NT * NX, RANK), t_grid, x_grid)
    return out[:, :RANK].reshape(b, n, RANK)
